# Initial kernel scaffold; baseline (speedup 1.0000x reference)
import jax
import jax.numpy as jnp
from jax.experimental import pallas as pl


def kernel(x, sa1_params, sa2_params, sa3_params, fp3_params, fp2_params, fp1_params, head_params, conv4_W, conv4_b):
    raise NotImplementedError("write your pallas kernel here")


# speedup vs baseline: 6.5271x; 6.5271x over previous
import jax
import jax.numpy as jnp
from jax.experimental import pallas as pl

F32 = jnp.float32
HI = jax.lax.Precision.HIGHEST
BIG = 1e9
B = 8
N0 = 8192
L = B * N0
CHUNK = 512
NCHUNK = N0 // CHUNK
COLS = 2048
NCOLS = L // COLS


def _iota(shape, dim):
    return jax.lax.broadcasted_iota(jnp.int32, shape, dim).astype(F32)


def _dot(a, b, ca, cb, batch=False, prec=None):
    bd = (((0,), (0,)) if batch else ((), ()))
    return jax.lax.dot_general(a, b, (((ca,), (cb,)), bd), precision=prec)


def _sumsq_ch(xc):
    acc = xc[:, 0, :] * xc[:, 0, :]
    for c in range(1, xc.shape[1]):
        acc = acc + xc[:, c, :] * xc[:, c, :]
    return acc


def _sumsq_row(xr):
    acc = xr[..., 0] * xr[..., 0]
    for c in range(1, xr.shape[-1]):
        acc = acc + xr[..., c] * xr[..., c]
    return acc


def _fps(xc, npoint):
    Bb, C, N = xc.shape
    iota_n = _iota((Bb, N), 1)
    iota_s = jax.lax.broadcasted_iota(jnp.int32, (Bb, npoint, C), 1)

    def body(i, carry):
        distance, far, cents = carry
        onehot = (iota_n == far).astype(F32)
        cent = _dot(onehot, xc, 1, 2, batch=True, prec=HI)
        cents = jnp.where(iota_s == i, cent[:, None, :], cents)
        d = (xc[:, 0, :] - cent[:, 0:1]) ** 2
        for c in range(1, C):
            d = d + (xc[:, c, :] - cent[:, c:c + 1]) ** 2
        distance = jnp.minimum(distance, d)
        mx = jnp.max(distance, axis=-1, keepdims=True)
        far = jnp.min(jnp.where(distance == mx, iota_n, BIG), axis=-1,
                      keepdims=True)
        return distance, far, cents

    init = (jnp.full((Bb, N), 1e10, F32), jnp.zeros((Bb, 1), F32),
            jnp.zeros((Bb, npoint, C), F32))
    _, _, cents = jax.lax.fori_loop(0, npoint, body, init)
    return cents


def _sq_dist(src_row, xc):
    d = -2.0 * _dot(src_row, xc, 2, 1, batch=True)
    d = d + _sumsq_row(src_row)[:, :, None]
    d = d + _sumsq_ch(xc)[:, None, :]
    return d


def _ball_query(radius, nsample, xc, cents):
    N = xc.shape[2]
    sq = _sq_dist(cents, xc)
    iota_n = _iota(sq.shape, 2)
    cur = jnp.where(sq > radius * radius, BIG, iota_n)
    idxs = []
    for _ in range(nsample):
        mn = jnp.min(cur, axis=-1, keepdims=True)
        idxs.append(mn)
        cur = jnp.where(iota_n == mn, BIG, cur)
    idx = jnp.concatenate(idxs, axis=-1)
    first = idx[..., 0:1]
    idx = jnp.where(idx >= BIG * 0.5, first, idx)
    return jnp.minimum(idx, float(N - 1))


def _gather_cn(xc, idx):
    Bb, C, N = xc.shape
    M = idx.shape[1]
    outs = []
    for b in range(Bb):
        oh = (idx[b][:, None] == _iota((M, N), 1)).astype(F32)
        outs.append(_dot(oh, xc[b], 1, 1, prec=HI))
    return jnp.stack(outs, axis=0)


def _mlp_bn(t, layers):
    for (W, bb, g, be) in layers:
        z = _dot(t, W, 2, 1) + bb[None]
        m = jnp.mean(z, axis=(0, 1), keepdims=True)
        v = jnp.var(z, axis=(0, 1), keepdims=True)
        t = jax.nn.relu((z - m) / jnp.sqrt(v + 1e-5) * g[None] + be[None])
    return t


def _top3(cur, iota_ax, axis):
    ds, idxs = [], []
    for _ in range(3):
        mn = jnp.min(cur, axis=axis, keepdims=True)
        am = jnp.min(jnp.where(cur == mn, iota_ax, BIG), axis=axis,
                     keepdims=True)
        ds.append(mn)
        idxs.append(am)
        cur = jnp.where(iota_ax == am, BIG, cur)
    return ds, idxs


def _interp_weights(ds):
    recips = [1.0 / (d + 1e-8) for d in ds]
    norm = (recips[0] + recips[1]) + recips[2]
    return [r / norm for r in recips]


def _kmajor(t4):
    Bb, S, K, C = t4.shape
    return jnp.transpose(t4, (0, 2, 1, 3)).reshape(Bb, K * S, C)


def _stage_a(*refs):
    x_ref = refs[0]
    cents_out, l1cm_out = refs[-2], refs[-1]
    params = refs[1:-2]
    cursor = [0]

    def take(nlayers):
        out = []
        for _ in range(nlayers):
            i = cursor[0]
            out.append((params[i][...], params[i + 1][...],
                        params[i + 2][...], params[i + 3][...]))
            cursor[0] += 4
        return out

    sa1, sa2, sa3, fp3, fp2 = take(3), take(3), take(3), take(2), take(2)
    x = x_ref[...]

    cents1 = _fps(x, 32)
    idx1 = _ball_query(0.2, 8, x, cents1)
    g1 = _gather_cn(x, idx1.reshape(B, 256))
    g1 = g1.reshape(B, 32, 8, 5)
    grouped = jnp.concatenate([g1 - cents1[:, :, None, :], g1], axis=-1)
    t1 = _mlp_bn(_kmajor(grouped), sa1)
    l1_points = jnp.max(t1.reshape(B, 8, 32, 128), axis=1)

    cents1_cm = jnp.transpose(cents1, (0, 2, 1))
    l1_points_cm = jnp.transpose(l1_points, (0, 2, 1))
    feat1_cm = jnp.concatenate([cents1_cm, l1_points_cm], axis=1)
    cents2 = _fps(cents1_cm, 16)
    idx2 = _ball_query(0.4, 8, cents1_cm, cents2)
    g2 = _gather_cn(feat1_cm, idx2.reshape(B, 128))
    g2 = g2.reshape(B, 16, 8, 133)
    gxyz2 = g2[..., :5] - cents2[:, :, None, :]
    grouped2 = jnp.concatenate([gxyz2, g2[..., 5:]], axis=-1)
    t2 = _mlp_bn(_kmajor(grouped2), sa2)
    l2_points = jnp.max(t2.reshape(B, 8, 16, 256), axis=1)

    t3 = jnp.concatenate([cents2, l2_points], axis=-1)
    t3 = _mlp_bn(t3, sa3)
    l3_points = jnp.max(t3, axis=1)

    interp3 = jnp.broadcast_to(l3_points[:, None, :], (B, 16, 512))
    t4 = jnp.concatenate([l2_points, interp3], axis=-1)
    l2_new = _mlp_bn(t4, fp3)

    cents2_cm = jnp.transpose(cents2, (0, 2, 1))
    d2 = _sq_dist(cents1, cents2_cm)
    iota16 = _iota(d2.shape, 2)
    ds, idxs = _top3(d2, iota16, axis=-1)
    ws = _interp_weights(ds)
    l2_new_cm = jnp.transpose(l2_new, (0, 2, 1))
    gk = [_gather_cn(l2_new_cm, idxs[k][..., 0]) for k in range(3)]
    interp2 = ((gk[0] * ws[0] + gk[1] * ws[1]) + gk[2] * ws[2])
    t5 = jnp.concatenate([l1_points, interp2], axis=-1)
    l1_new = _mlp_bn(t5, fp2)

    cents_out[...] = cents1
    l1cm_out[...] = jnp.transpose(l1_new, (0, 2, 1))


def _interp_body(x_ref, cents_ref, l1_ref, W_ref, b_ref, z_ref):
    xb = x_ref[0]
    cents = cents_ref[0]
    l1 = l1_ref[0]
    d = -2.0 * _dot(cents, xb, 1, 0)
    d = d + _sumsq_ch(xb[None])[0][None, :]
    d = d + _sumsq_row(cents[None])[0][:, None]
    iota32 = _iota(d.shape, 0)
    ds, idxs = _top3(d, iota32, axis=0)
    ws = _interp_weights(ds)
    gk = []
    for k in range(3):
        oh = (iota32 == idxs[k]).astype(F32)
        gk.append(_dot(l1, oh, 1, 0, prec=HI))
    interp = (gk[0] * ws[0] + gk[1] * ws[1]) + gk[2] * ws[2]
    z_ref[...] = _dot(W_ref[...], interp, 1, 0) + b_ref[...]


def _stats_body(z_ref, m_ref, v_ref):
    z = z_ref[...]
    m_ref[...] = jnp.mean(z, axis=1, keepdims=True)
    v_ref[...] = jnp.var(z, axis=1, keepdims=True)


def _stats(z, cblk):
    C = z.shape[0]
    return pl.pallas_call(
        _stats_body,
        grid=(C // cblk,),
        in_specs=[pl.BlockSpec((cblk, L), lambda i: (i, 0))],
        out_specs=(pl.BlockSpec((cblk, 1), lambda i: (i, 0)),
                   pl.BlockSpec((cblk, 1), lambda i: (i, 0))),
        out_shape=(jax.ShapeDtypeStruct((C, 1), F32),
                   jax.ShapeDtypeStruct((C, 1), F32)),
    )(z)


def _layer_body(z_ref, m_ref, v_ref, g_ref, be_ref, W_ref, b_ref, zn_ref):
    z = z_ref[...]
    m = m_ref[...]
    v = v_ref[...]
    a = jax.nn.relu((z - m) / jnp.sqrt(v + 1e-5) * g_ref[...] + be_ref[...])
    zn_ref[...] = _dot(W_ref[...], a, 1, 0) + b_ref[...]


def _final_body(z_ref, m_ref, v_ref, g_ref, be_ref, W_ref, b_ref, out_ref):
    z = z_ref[...]
    m = m_ref[...]
    v = v_ref[...]
    a = jax.nn.relu((z - m) / jnp.sqrt(v + 1e-5) * g_ref[...] + be_ref[...])
    out_ref[0] = _dot(W_ref[...], a, 1, 0) + b_ref[...]


def kernel(x, sa1_params, sa2_params, sa3_params, fp3_params, fp2_params,
           fp1_params, head_params, conv4_W, conv4_b):
    flat = []
    for layers in (sa1_params, sa2_params, sa3_params, fp3_params,
                   fp2_params):
        for (W, bb, g, be) in layers:
            flat += [W, bb.reshape(1, -1), g.reshape(1, -1), be.reshape(1, -1)]

    cents1, l1cm = pl.pallas_call(
        _stage_a,
        out_shape=(jax.ShapeDtypeStruct((B, 32, 5), F32),
                   jax.ShapeDtypeStruct((B, 128, 32), F32)),
    )(x, *flat)

    bn_layers = list(fp1_params) + list(head_params)
    W1, b1 = bn_layers[0][0], bn_layers[0][1]
    z = pl.pallas_call(
        _interp_body,
        grid=(B, NCHUNK),
        in_specs=[pl.BlockSpec((1, 5, CHUNK), lambda i, j: (i, 0, j)),
                  pl.BlockSpec((1, 32, 5), lambda i, j: (i, 0, 0)),
                  pl.BlockSpec((1, 128, 32), lambda i, j: (i, 0, 0)),
                  pl.BlockSpec((128, 128), lambda i, j: (0, 0)),
                  pl.BlockSpec((128, 1), lambda i, j: (0, 0))],
        out_specs=pl.BlockSpec((128, CHUNK),
                               lambda i, j: (0, i * NCHUNK + j)),
        out_shape=jax.ShapeDtypeStruct((128, L), F32),
    )(x, cents1, l1cm, W1, b1.reshape(128, 1))

    for li in range(1, 6):
        g_prev, be_prev = bn_layers[li - 1][2], bn_layers[li - 1][3]
        Wn, bn_ = bn_layers[li][0], bn_layers[li][1]
        cin, cout = Wn.shape[1], Wn.shape[0]
        m, v = _stats(z, 64 if cin % 64 == 0 else cin)
        z = pl.pallas_call(
            _layer_body,
            grid=(NCOLS,),
            in_specs=[pl.BlockSpec((cin, COLS), lambda i: (0, i))] +
                     [pl.BlockSpec((cin, 1), lambda i: (0, 0))] * 4 +
                     [pl.BlockSpec((cout, cin), lambda i: (0, 0)),
                      pl.BlockSpec((cout, 1), lambda i: (0, 0))],
            out_specs=pl.BlockSpec((cout, COLS), lambda i: (0, i)),
            out_shape=jax.ShapeDtypeStruct((cout, L), F32),
        )(z, m, v, g_prev.reshape(cin, 1), be_prev.reshape(cin, 1), Wn,
          bn_.reshape(cout, 1))

    g_prev, be_prev = bn_layers[5][2], bn_layers[5][3]
    m, v = _stats(z, 25)
    out = pl.pallas_call(
        _final_body,
        grid=(B, NCHUNK),
        in_specs=[pl.BlockSpec((25, CHUNK), lambda i, j: (0, i * NCHUNK + j))]
                 + [pl.BlockSpec((25, 1), lambda i, j: (0, 0))] * 4 +
                 [pl.BlockSpec((128, 25), lambda i, j: (0, 0)),
                  pl.BlockSpec((128, 1), lambda i, j: (0, 0))],
        out_specs=pl.BlockSpec((1, 128, CHUNK), lambda i, j: (i, 0, j)),
        out_shape=jax.ShapeDtypeStruct((B, 128, N0), F32),
    )(z, m, v, g_prev.reshape(25, 1), be_prev.reshape(25, 1), conv4_W,
      conv4_b.reshape(128, 1))
    return out
